# baseline (speedup 1.0000x reference)
import functools

import jax
import jax.numpy as jnp
from jax import lax
from jax.experimental import pallas as pl
from jax.experimental.pallas import tpu as pltpu
from jax.experimental.pallas import tpu_sc as plsc

NUM_GROUPS = 2
NUM_VARS = 320
ROW_BLOCK = 256

_NC, _NS = 1, 16
_NW = _NC * _NS


def _tc_body(x_ref, w_ref, b_ref, idx0_ref, idx1_ref, ppl_ref, counts_ref):
    i = pl.program_id(0)
    n = pl.num_programs(0)

    @pl.when(i == 0)
    def _():
        counts_ref[...] = jnp.zeros_like(counts_ref)

    x = x_ref[...]
    rows = x.shape[0]
    logits = jnp.dot(x, w_ref[...], preferred_element_type=jnp.float32)
    logits = logits + b_ref[...]
    colf = lax.broadcasted_iota(jnp.int32, (rows, NUM_VARS), 1).astype(jnp.float32)
    idx_refs = (idx0_ref, idx1_ref)
    for g in range(NUM_GROUPS):
        lg = lax.slice(logits, (0, g * NUM_VARS), (rows, (g + 1) * NUM_VARS))
        m = jnp.max(lg, axis=1, keepdims=True)
        idxf = jnp.min(jnp.where(lg == m, colf, jnp.float32(NUM_VARS)), axis=1)
        onehot = (colf == idxf[:, None]).astype(jnp.float32)
        counts_ref[g : g + 1, :] += jnp.sum(onehot, axis=0, keepdims=True)
        idx_refs[g][...] = idxf.astype(jnp.int32) + g * NUM_VARS

    @pl.when(i == n - 1)
    def _():
        total = jnp.float32(n * rows)
        p = counts_ref[0:NUM_GROUPS, :] / total
        neg_ent = jnp.sum(p * jnp.log(p + 1e-7), axis=1)
        ppl_ref[0, 0] = jnp.sum(jnp.exp(-neg_ent))


def _tc_quantize(x2d, w, b1d):
    n_rows = x2d.shape[0]
    grid = (n_rows // ROW_BLOCK,)
    return pl.pallas_call(
        _tc_body,
        grid=grid,
        in_specs=[
            pl.BlockSpec((ROW_BLOCK, x2d.shape[1]), lambda i: (i, 0)),
            pl.BlockSpec(w.shape, lambda i: (0, 0)),
            pl.BlockSpec(b1d.shape, lambda i: (0,)),
        ],
        out_specs=[
            pl.BlockSpec((ROW_BLOCK,), lambda i: (i,)),
            pl.BlockSpec((ROW_BLOCK,), lambda i: (i,)),
            pl.BlockSpec(memory_space=pltpu.SMEM),
        ],
        out_shape=[
            jax.ShapeDtypeStruct((n_rows,), jnp.int32),
            jax.ShapeDtypeStruct((n_rows,), jnp.int32),
            jax.ShapeDtypeStruct((1, 1), jnp.float32),
        ],
        scratch_shapes=[pltpu.VMEM((8, NUM_VARS), jnp.float32)],
        compiler_params=pltpu.CompilerParams(
            dimension_semantics=("arbitrary",),
        ),
    )(x2d, w, b1d)


def _sc_combine(table, idx0, idx1):
    n_rows = idx0.shape[0]
    d = table.shape[1]
    r_per_w = n_rows // _NW
    mesh = plsc.VectorSubcoreMesh(core_axis_name="c", subcore_axis_name="s", num_cores=1)

    @functools.partial(
        pl.kernel,
        mesh=mesh,
        out_type=jax.ShapeDtypeStruct((n_rows, NUM_GROUPS * d), jnp.float32),
        scratch_types=[
            pltpu.VMEM((r_per_w,), jnp.int32),
            pltpu.VMEM((r_per_w,), jnp.int32),
            pltpu.VMEM((r_per_w, d), jnp.float32),
            pltpu.VMEM((r_per_w, d), jnp.float32),
            pltpu.SemaphoreType.DMA,
            pltpu.SemaphoreType.DMA,
            pltpu.SemaphoreType.DMA,
        ],
    )
    def k(table_hbm, idx0_hbm, idx1_hbm, out_hbm, idx0_v, idx1_v, rows0_v, rows1_v, sem_i, sem_g, sem_w):
        wid = lax.axis_index("s") * _NC + lax.axis_index("c")
        base = wid * r_per_w
        ld0 = pltpu.async_copy(idx0_hbm.at[pl.ds(base, r_per_w)], idx0_v, sem_i)
        ld1 = pltpu.async_copy(idx1_hbm.at[pl.ds(base, r_per_w)], idx1_v, sem_i)
        ld0.wait()
        g0 = pltpu.async_copy(table_hbm.at[idx0_v], rows0_v, sem_g)
        ld1.wait()
        g1 = pltpu.async_copy(table_hbm.at[idx1_v], rows1_v, sem_g)
        g0.wait()
        w0 = pltpu.async_copy(
            rows0_v, out_hbm.at[pl.ds(base, r_per_w), pl.ds(0, d)], sem_w
        )
        g1.wait()
        w1 = pltpu.async_copy(
            rows1_v, out_hbm.at[pl.ds(base, r_per_w), pl.ds(d, d)], sem_w
        )
        w0.wait()
        w1.wait()

    return k(table, idx0, idx1)


def kernel(hidden_states, W_proj, b_proj, codevectors):
    batch, seq, hidden = hidden_states.shape
    x2d = hidden_states.reshape(batch * seq, hidden)
    idx0, idx1, ppl = _tc_quantize(x2d, W_proj, b_proj)

    d = codevectors.shape[-1]
    table = codevectors.reshape(NUM_GROUPS * NUM_VARS, d)
    cv2d = _sc_combine(table, idx0, idx1)
    cv = cv2d.reshape(batch, seq, NUM_GROUPS * d)
    return cv, ppl.reshape(())


# speedup vs baseline: 1.0835x; 1.0835x over previous
import functools

import jax
import jax.numpy as jnp
from jax import lax
from jax.experimental import pallas as pl
from jax.experimental.pallas import tpu as pltpu
from jax.experimental.pallas import tpu_sc as plsc

NUM_GROUPS = 2
NUM_VARS = 320
ROW_BLOCK = 512

_NC, _NS = 1, 16
_NW = _NC * _NS


def _tc_body(x_ref, w_ref, b_ref, idx0_ref, idx1_ref, ppl_ref, counts_ref):
    i = pl.program_id(0)
    n = pl.num_programs(0)

    @pl.when(i == 0)
    def _():
        counts_ref[...] = jnp.zeros_like(counts_ref)

    x = x_ref[...]
    rows = x.shape[0]
    logits = jnp.dot(x, w_ref[...], preferred_element_type=jnp.float32)
    logits = logits + b_ref[...]
    colf = lax.broadcasted_iota(jnp.int32, (rows, NUM_VARS), 1).astype(jnp.float32)
    idx_refs = (idx0_ref, idx1_ref)
    for g in range(NUM_GROUPS):
        lg = lax.slice(logits, (0, g * NUM_VARS), (rows, (g + 1) * NUM_VARS))
        m = jnp.max(lg, axis=1, keepdims=True)
        idxf = jnp.min(jnp.where(lg == m, colf, jnp.float32(NUM_VARS)), axis=1)
        onehot = (colf == idxf[:, None]).astype(jnp.float32)
        counts_ref[g : g + 1, :] += jnp.sum(onehot, axis=0, keepdims=True)
        idx_refs[g][...] = idxf.astype(jnp.int32) + g * NUM_VARS

    @pl.when(i == n - 1)
    def _():
        total = jnp.float32(n * rows)
        p = counts_ref[0:NUM_GROUPS, :] / total
        neg_ent = jnp.sum(p * jnp.log(p + 1e-7), axis=1)
        ppl_ref[0, 0] = jnp.sum(jnp.exp(-neg_ent))


def _tc_quantize(x2d, w, b1d):
    n_rows = x2d.shape[0]
    grid = (n_rows // ROW_BLOCK,)
    return pl.pallas_call(
        _tc_body,
        grid=grid,
        in_specs=[
            pl.BlockSpec((ROW_BLOCK, x2d.shape[1]), lambda i: (i, 0)),
            pl.BlockSpec(w.shape, lambda i: (0, 0)),
            pl.BlockSpec(b1d.shape, lambda i: (0,)),
        ],
        out_specs=[
            pl.BlockSpec((ROW_BLOCK,), lambda i: (i,)),
            pl.BlockSpec((ROW_BLOCK,), lambda i: (i,)),
            pl.BlockSpec(memory_space=pltpu.SMEM),
        ],
        out_shape=[
            jax.ShapeDtypeStruct((n_rows,), jnp.int32),
            jax.ShapeDtypeStruct((n_rows,), jnp.int32),
            jax.ShapeDtypeStruct((1, 1), jnp.float32),
        ],
        scratch_shapes=[pltpu.VMEM((8, NUM_VARS), jnp.float32)],
        compiler_params=pltpu.CompilerParams(
            dimension_semantics=("arbitrary",),
        ),
    )(x2d, w, b1d)


def _sc_combine(table, idx0, idx1):
    n_rows = idx0.shape[0]
    d = table.shape[1]
    r_per_w = n_rows // _NW
    mesh = plsc.VectorSubcoreMesh(core_axis_name="c", subcore_axis_name="s", num_cores=1)

    half = _NW // NUM_GROUPS
    n_per_w = n_rows // half

    @functools.partial(
        pl.kernel,
        mesh=mesh,
        out_type=jax.ShapeDtypeStruct((n_rows, NUM_GROUPS * d), jnp.float32),
        scratch_types=[
            pltpu.VMEM((n_per_w,), jnp.int32),
            pltpu.VMEM((n_per_w, d), jnp.float32),
            pltpu.SemaphoreType.DMA,
        ],
    )
    def k(table_hbm, idx0_hbm, idx1_hbm, out_hbm, idx_v, rows_v, sem_g):
        wid = lax.axis_index("s") * _NC + lax.axis_index("c")
        grp = wid // half
        base = (wid % half) * n_per_w
        col = grp * d

        @pl.when(grp == 0)
        def _():
            pltpu.sync_copy(idx0_hbm.at[pl.ds(base, n_per_w)], idx_v)

        @pl.when(grp == 1)
        def _():
            pltpu.sync_copy(idx1_hbm.at[pl.ds(base, n_per_w)], idx_v)

        pltpu.async_copy(table_hbm.at[idx_v], rows_v, sem_g).wait()
        pltpu.sync_copy(rows_v, out_hbm.at[pl.ds(base, n_per_w), pl.ds(col, d)])

    return k(table, idx0, idx1)


def kernel(hidden_states, W_proj, b_proj, codevectors):
    batch, seq, hidden = hidden_states.shape
    x2d = hidden_states.reshape(batch * seq, hidden)
    idx0, idx1, ppl = _tc_quantize(x2d, W_proj, b_proj)

    d = codevectors.shape[-1]
    table = codevectors.reshape(NUM_GROUPS * NUM_VARS, d)
    cv2d = _sc_combine(table, idx0, idx1)
    cv = cv2d.reshape(batch, seq, NUM_GROUPS * d)
    return cv, ppl.reshape(())
